# baseline (speedup 1.0000x reference)
import functools

import jax
import jax.numpy as jnp
from jax import lax
from jax.experimental import pallas as pl
from jax.experimental.pallas import tpu as pltpu
from jax.experimental.pallas import tpu_sc as plsc

N_NODE = 40727
EMB = 100
LAYERS = 3
B = 1024
L = 50
NNZ = 651632
DP = 128
VB = 512
NVP = 40960
BB = 256

NW = 32
NB = 128
R = 320
TROWS = NB * R
ZROW = N_NODE
KE = 128
NNZP = 651776
SK = 80

_f32 = jnp.float32
_i32 = jnp.int32


def _wid():
    return lax.axis_index("s") * 2 + lax.axis_index("c")


def _spmm_body(table, colslr, vals, bounds, zeros, out,
               acc, gath, ibuf, vbuf, gidx, bounds_v, sem_i, sem_g):
    w = _wid()
    pltpu.sync_copy(bounds, bounds_v)

    bw = bounds_v[w]

    def fire_idx(off, slot):
        pltpu.async_copy(colslr.at[pl.ds(off, KE)], ibuf.at[slot],
                         sem_i.at[slot])
        pltpu.async_copy(vals.at[pl.ds(off, KE)], vbuf.at[slot],
                         sem_i.at[slot])

    def wait_idx(off, slot):
        pltpu.make_async_copy(colslr.at[pl.ds(off, KE)], ibuf.at[slot],
                              sem_i.at[slot]).wait()
        pltpu.make_async_copy(vals.at[pl.ds(off, KE)], vbuf.at[slot],
                              sem_i.at[slot]).wait()

    def unpack_and_gather(slot, par):
        def up(g, _):
            sl = pl.ds(g * 16, 16)
            gidx[par, sl] = jnp.bitwise_and(ibuf[slot, sl], 0xFFFF)
            return 0
        lax.fori_loop(0, KE // 16, up, 0)
        pltpu.async_copy(table.at[gidx.at[par]], gath.at[par],
                         sem_g.at[par])

    def wait_gather(par):
        pltpu.make_async_copy(table.at[gidx.at[par]], gath.at[par],
                              sem_g.at[par]).wait()

    def do_block(k):
        b = w * (NB // NW) + k
        e0 = bw[k]
        e1 = bw[k + 1]
        ea = (e0 // 8) * 8
        nchunks = (e1 - ea + (KE - 1)) // KE
        pltpu.sync_copy(zeros, acc)

        @pl.when(nchunks > 0)
        def _():
            fire_idx(ea, 0)

        @pl.when(nchunks > 1)
        def _():
            fire_idx(ea + KE, 1)

        @pl.when(nchunks > 0)
        def _():
            wait_idx(ea, 0)
            unpack_and_gather(0, 0)

        def chunk_body(j, _):
            off = ea + j * KE
            jp1 = j + 1
            jp2 = j + 2
            i1 = lax.rem(jp1, 3)
            i2 = lax.rem(jp2, 3)
            i0 = lax.rem(j, 3)
            p0 = lax.rem(j, 2)
            p1 = lax.rem(jp1, 2)

            @pl.when(jp2 < nchunks)
            def _():
                fire_idx(ea + jp2 * KE, i2)

            @pl.when(jp1 < nchunks)
            def _():
                wait_idx(ea + jp1 * KE, i1)
                unpack_and_gather(i1, p1)

            wait_gather(p0)

            i0full = jnp.full((16,), i0, _i32)

            def group_body(g, _):
                sl = pl.ds(g * 16, 16)
                e16 = lax.iota(_i32, 16) + (off + g * 16)
                ok16 = jnp.logical_and(e16 >= e0, e16 < e1)
                vbuf[i0, sl] = jnp.where(ok16, vbuf[i0, sl], 0.0)
                for i in range(16):
                    e = g * 16 + i
                    evec = jnp.full((16,), e, _i32)
                    v = plsc.load_gather(vbuf, [i0full, evec])
                    lrb = jnp.right_shift(
                        plsc.load_gather(ibuf, [i0full, evec]), 16)
                    for d in range(DP // 16):
                        dsl = pl.ds(d * 16, 16)
                        cidx = lax.iota(_i32, 16) + d * 16
                        plsc.addupdate_scatter(
                            acc, [lrb, cidx], gath[p0, e, dsl] * v)
                return 0

            lax.fori_loop(0, KE // 16, group_body, 0)
            return 0

        lax.fori_loop(0, nchunks, chunk_body, 0)
        base = b * R
        pltpu.sync_copy(acc, out.at[pl.ds(base, R)])

    for k in range(NB // NW):
        do_block(k)


@functools.partial(
    pl.kernel,
    out_type=jax.ShapeDtypeStruct((TROWS, DP), _f32),
    mesh=plsc.VectorSubcoreMesh(core_axis_name="c", subcore_axis_name="s",
                                num_cores=2, num_subcores=16),
    compiler_params=pltpu.CompilerParams(needs_layout_passes=False),
    scratch_types=[
        pltpu.VMEM((R, DP), _f32),
        pltpu.VMEM((2, KE, DP), _f32),
        pltpu.VMEM((3, KE), _i32),
        pltpu.VMEM((3, KE), _f32),
        pltpu.VMEM((2, KE), _i32),
        pltpu.VMEM((NW, 16), _i32),
        pltpu.SemaphoreType.DMA((3,)),
        pltpu.SemaphoreType.DMA((2,)),
    ],
)
def _spmm(table, colslr, vals, bounds, zeros, out,
          acc, gath, ibuf, vbuf, gidx, bounds_v, sem_i, sem_g):
    _spmm_body(table, colslr, vals, bounds, zeros, out,
               acc, gath, ibuf, vbuf, gidx, bounds_v, sem_i, sem_g)


def _gather4_body(t0, t1, t2, t3, idx, out,
                  idx_v, g0, g1, g2, g3, ov, sem):
    w = _wid()
    per_w = (L * B) // NW
    nchunks = per_w // SK

    def chunk(j, _):
        off = w * per_w + j * SK
        pltpu.sync_copy(idx.at[pl.ds(off, SK)], idx_v)
        c0 = pltpu.async_copy(t0.at[idx_v], g0, sem)
        c1 = pltpu.async_copy(t1.at[idx_v], g1, sem)
        c2 = pltpu.async_copy(t2.at[idx_v], g2, sem)
        c3 = pltpu.async_copy(t3.at[idx_v], g3, sem)
        c0.wait(); c1.wait(); c2.wait(); c3.wait()

        def row(i, _):
            for d in range(DP // 16):
                sl = pl.ds(d * 16, 16)
                ov[i, sl] = (g0[i, sl] + g1[i, sl] + g2[i, sl]
                             + g3[i, sl]) * 0.25
            return 0

        lax.fori_loop(0, SK, row, 0)
        pltpu.sync_copy(ov, out.at[pl.ds(off, SK)])
        return 0

    lax.fori_loop(0, nchunks, chunk, 0)


@functools.partial(
    pl.kernel,
    out_type=jax.ShapeDtypeStruct((L * B, DP), _f32),
    mesh=plsc.VectorSubcoreMesh(core_axis_name="c", subcore_axis_name="s",
                                num_cores=2, num_subcores=16),
    scratch_types=[
        pltpu.VMEM((SK,), _i32),
        pltpu.VMEM((SK, DP), _f32),
        pltpu.VMEM((SK, DP), _f32),
        pltpu.VMEM((SK, DP), _f32),
        pltpu.VMEM((SK, DP), _f32),
        pltpu.VMEM((SK, DP), _f32),
        pltpu.SemaphoreType.DMA,
    ],
)
def _gather4(t0, t1, t2, t3, idx, out, idx_v, g0, g1, g2, g3, ov, sem):
    _gather4_body(t0, t1, t2, t3, idx, out, idx_v, g0, g1, g2, g3, ov, sem)


def _attn_body(seq_ref, slen_ref, w1_ref, b1_ref, w2_ref, f_ref, theta_ref):
    z = seq_ref[0]
    for l in range(1, L):
        z = z + seq_ref[l]
    xs = z / slen_ref[...]
    h = jnp.dot(xs, w2_ref[...], preferred_element_type=jnp.float32)
    b1 = b1_ref[...]
    fr = f_ref[...]
    theta = jnp.zeros_like(h)
    for l in range(L):
        s = seq_ref[l]
        g = jnp.dot(jnp.tanh(s), w1_ref[...], preferred_element_type=jnp.float32)
        xt = jax.nn.sigmoid(g + b1 + h)
        beta = jnp.sum(xt * fr, axis=1, keepdims=True)
        theta = theta + beta * s
    theta_ref[...] = theta


def _attention(seq_t, slen, w1p, b1p, w2p, fp):
    grid = (B // BB,)
    return pl.pallas_call(
        _attn_body,
        grid=grid,
        in_specs=[
            pl.BlockSpec((L, BB, DP), lambda i: (0, i, 0)),
            pl.BlockSpec((BB, 1), lambda i: (i, 0)),
            pl.BlockSpec((DP, DP), lambda i: (0, 0)),
            pl.BlockSpec((1, DP), lambda i: (0, 0)),
            pl.BlockSpec((DP, DP), lambda i: (0, 0)),
            pl.BlockSpec((1, DP), lambda i: (0, 0)),
        ],
        out_specs=pl.BlockSpec((BB, DP), lambda i: (i, 0)),
        out_shape=jax.ShapeDtypeStruct((B, DP), jnp.float32),
    )(seq_t, slen, w1p, b1p, w2p, fp)


def _scores_body(theta_ref, emb_ref, out_ref):
    out_ref[...] = jax.lax.dot_general(
        theta_ref[...], emb_ref[...],
        (((1,), (1,)), ((), ())),
        preferred_element_type=jnp.float32)


def _scores(theta, emb_m):
    grid = (NVP // VB,)
    return pl.pallas_call(
        _scores_body,
        grid=grid,
        in_specs=[
            pl.BlockSpec((B, DP), lambda j: (0, 0)),
            pl.BlockSpec((VB, DP), lambda j: (j, 0)),
        ],
        out_specs=pl.BlockSpec((B, VB), lambda j: (0, j)),
        out_shape=jax.ShapeDtypeStruct((B, NVP), jnp.float32),
    )(theta, emb_m)


def kernel(adj_rows, adj_cols, adj_vals, session_len, session_item,
           reversed_sess_item, mask, A, D, embedding, glu1_W, glu1_b,
           glu2_W, f):
    colslr = jnp.pad(
        jnp.bitwise_or(adj_cols, jnp.left_shift(adj_rows % R, 16)),
        (0, NNZP - NNZ))
    rowsp = jnp.pad(adj_rows, (0, NNZP - NNZ), constant_values=NB * R)
    valsp = jnp.pad(adj_vals, (0, NNZP - NNZ))
    bnd = jnp.searchsorted(rowsp, jnp.arange(NB + 1, dtype=_i32) * R)
    bidx = jnp.clip(jnp.arange(NW)[:, None] * (NB // NW)
                    + jnp.arange(16)[None, :], 0, NB)
    bounds = bnd.astype(_i32)[bidx]
    zeros = jnp.zeros((R, DP), _f32)

    t0 = jnp.zeros((TROWS, DP), _f32).at[:N_NODE, :EMB].set(embedding)

    t1 = _spmm(t0, colslr, valsp, bounds, zeros)
    t2 = _spmm(t1, colslr, valsp, bounds, zeros)
    t3 = _spmm(t2, colslr, valsp, bounds, zeros)

    rev_t = jnp.reshape(jnp.transpose(reversed_sess_item), (L * B,))
    idx_t = jnp.where(rev_t == 0, ZROW, rev_t - 1).astype(_i32)
    seq_flat = _gather4(t0, t1, t2, t3, idx_t)
    seq_t = jnp.reshape(seq_flat, (L, B, DP))

    w1p = jnp.pad(glu1_W, ((0, DP - EMB), (0, DP - EMB)))
    b1p = jnp.pad(glu1_b, (0, DP - EMB)).reshape(1, DP)
    w2p = jnp.pad(glu2_W, ((0, DP - EMB), (0, DP - EMB)))
    fp = jnp.pad(f[:, 0], (0, DP - EMB)).reshape(1, DP)

    theta = _attention(seq_t, session_len, w1p, b1p, w2p, fp)

    emb_m = jnp.pad(embedding, ((0, NVP - N_NODE), (0, DP - EMB)))
    scores = _scores(theta, emb_m)
    return scores[:, :N_NODE]


# speedup vs baseline: 2.2170x; 2.2170x over previous
import functools

import jax
import jax.numpy as jnp
from jax import lax
from jax.experimental import pallas as pl
from jax.experimental.pallas import tpu as pltpu
from jax.experimental.pallas import tpu_sc as plsc

N_NODE = 40727
EMB = 100
LAYERS = 3
B = 1024
L = 50
NNZ = 651632
DP = 128
VB = 512
NVP = 40960
BB = 256

NW = 32
NB = 128
R = 320
TROWS = NB * R
ZROW = N_NODE
KE = 128
NNZP = 651776
SK = 80

_f32 = jnp.float32
_i32 = jnp.int32


def _wid():
    return lax.axis_index("s") * 2 + lax.axis_index("c")


def _spmm_body(table, colslr, vals, bounds, zeros, out,
               acc, gath, ibuf, vbuf, gidx, bounds_v, sem_i, sem_g):
    w = _wid()
    pltpu.sync_copy(bounds, bounds_v)

    bw = bounds_v[w]

    def fire_idx(off, slot):
        pltpu.async_copy(colslr.at[pl.ds(off, KE)], ibuf.at[slot],
                         sem_i.at[slot])
        pltpu.async_copy(vals.at[pl.ds(off, KE)], vbuf.at[slot],
                         sem_i.at[slot])

    def wait_idx(off, slot):
        pltpu.make_async_copy(colslr.at[pl.ds(off, KE)], ibuf.at[slot],
                              sem_i.at[slot]).wait()
        pltpu.make_async_copy(vals.at[pl.ds(off, KE)], vbuf.at[slot],
                              sem_i.at[slot]).wait()

    def unpack_and_gather(slot, par):
        def up(g, _):
            sl = pl.ds(g * 16, 16)
            gidx[par, sl] = jnp.bitwise_and(ibuf[slot, sl], 0xFFFF)
            return 0
        lax.fori_loop(0, KE // 16, up, 0)
        pltpu.async_copy(table.at[gidx.at[par]], gath.at[par],
                         sem_g.at[par])

    def wait_gather(par):
        pltpu.make_async_copy(table.at[gidx.at[par]], gath.at[par],
                              sem_g.at[par]).wait()

    def do_block(k):
        b = w * (NB // NW) + k
        e0 = bw[k]
        e1 = bw[k + 1]
        ea = (e0 // 8) * 8
        nchunks = (e1 - ea + (KE - 1)) // KE
        pltpu.sync_copy(zeros, acc)

        @pl.when(nchunks > 0)
        def _():
            fire_idx(ea, 0)

        @pl.when(nchunks > 1)
        def _():
            fire_idx(ea + KE, 1)

        @pl.when(nchunks > 0)
        def _():
            wait_idx(ea, 0)
            unpack_and_gather(0, 0)

        def chunk_body(j, _):
            off = ea + j * KE
            jp1 = j + 1
            jp2 = j + 2
            i1 = lax.rem(jp1, 3)
            i2 = lax.rem(jp2, 3)
            i0 = lax.rem(j, 3)
            p0 = lax.rem(j, 2)
            p1 = lax.rem(jp1, 2)

            @pl.when(jp2 < nchunks)
            def _():
                fire_idx(ea + jp2 * KE, i2)

            @pl.when(jp1 < nchunks)
            def _():
                wait_idx(ea + jp1 * KE, i1)
                unpack_and_gather(i1, p1)

            wait_gather(p0)

            i0full = jnp.full((16,), i0, _i32)

            def group_body(g, _):
                sl = pl.ds(g * 16, 16)
                e16 = lax.iota(_i32, 16) + (off + g * 16)
                ok16 = jnp.logical_and(e16 >= e0, e16 < e1)
                vbuf[i0, sl] = jnp.where(ok16, vbuf[i0, sl], 0.0)
                for i in range(16):
                    e = g * 16 + i
                    evec = jnp.full((16,), e, _i32)
                    v = plsc.load_gather(vbuf, [i0full, evec])
                    lrb = jnp.right_shift(
                        plsc.load_gather(ibuf, [i0full, evec]), 16)
                    prods = [gath[p0, e, pl.ds(d * 16, 16)] * v
                             for d in range(DP // 16)]
                    for d in range(DP // 16):
                        cidx = lax.iota(_i32, 16) + d * 16
                        plsc.addupdate_scatter(acc, [lrb, cidx], prods[d])
                return 0

            lax.fori_loop(0, KE // 16, group_body, 0)
            return 0

        lax.fori_loop(0, nchunks, chunk_body, 0)
        base = b * R
        pltpu.sync_copy(acc, out.at[pl.ds(base, R)])

    for k in range(NB // NW):
        do_block(k)


@functools.partial(
    pl.kernel,
    out_type=jax.ShapeDtypeStruct((TROWS, DP), _f32),
    mesh=plsc.VectorSubcoreMesh(core_axis_name="c", subcore_axis_name="s",
                                num_cores=2, num_subcores=16),
    compiler_params=pltpu.CompilerParams(needs_layout_passes=False),
    scratch_types=[
        pltpu.VMEM((R, DP), _f32),
        pltpu.VMEM((2, KE, DP), _f32),
        pltpu.VMEM((3, KE), _i32),
        pltpu.VMEM((3, KE), _f32),
        pltpu.VMEM((2, KE), _i32),
        pltpu.VMEM((NW, 16), _i32),
        pltpu.SemaphoreType.DMA((3,)),
        pltpu.SemaphoreType.DMA((2,)),
    ],
)
def _spmm(table, colslr, vals, bounds, zeros, out,
          acc, gath, ibuf, vbuf, gidx, bounds_v, sem_i, sem_g):
    _spmm_body(table, colslr, vals, bounds, zeros, out,
               acc, gath, ibuf, vbuf, gidx, bounds_v, sem_i, sem_g)


def _gather4_body(t0, t1, t2, t3, idx, out,
                  idx_v, g0, g1, g2, g3, ov, sem):
    w = _wid()
    per_w = (L * B) // NW
    nchunks = per_w // SK

    def chunk(j, _):
        off = w * per_w + j * SK
        pltpu.sync_copy(idx.at[pl.ds(off, SK)], idx_v)
        c0 = pltpu.async_copy(t0.at[idx_v], g0, sem)
        c1 = pltpu.async_copy(t1.at[idx_v], g1, sem)
        c2 = pltpu.async_copy(t2.at[idx_v], g2, sem)
        c3 = pltpu.async_copy(t3.at[idx_v], g3, sem)
        c0.wait(); c1.wait(); c2.wait(); c3.wait()

        def row(i, _):
            for d in range(DP // 16):
                sl = pl.ds(d * 16, 16)
                ov[i, sl] = (g0[i, sl] + g1[i, sl] + g2[i, sl]
                             + g3[i, sl]) * 0.25
            return 0

        lax.fori_loop(0, SK, row, 0)
        pltpu.sync_copy(ov, out.at[pl.ds(off, SK)])
        return 0

    lax.fori_loop(0, nchunks, chunk, 0)


@functools.partial(
    pl.kernel,
    out_type=jax.ShapeDtypeStruct((L * B, DP), _f32),
    mesh=plsc.VectorSubcoreMesh(core_axis_name="c", subcore_axis_name="s",
                                num_cores=2, num_subcores=16),
    scratch_types=[
        pltpu.VMEM((SK,), _i32),
        pltpu.VMEM((SK, DP), _f32),
        pltpu.VMEM((SK, DP), _f32),
        pltpu.VMEM((SK, DP), _f32),
        pltpu.VMEM((SK, DP), _f32),
        pltpu.VMEM((SK, DP), _f32),
        pltpu.SemaphoreType.DMA,
    ],
)
def _gather4(t0, t1, t2, t3, idx, out, idx_v, g0, g1, g2, g3, ov, sem):
    _gather4_body(t0, t1, t2, t3, idx, out, idx_v, g0, g1, g2, g3, ov, sem)


def _attn_body(seq_ref, slen_ref, w1_ref, b1_ref, w2_ref, f_ref, theta_ref):
    z = seq_ref[0]
    for l in range(1, L):
        z = z + seq_ref[l]
    xs = z / slen_ref[...]
    h = jnp.dot(xs, w2_ref[...], preferred_element_type=jnp.float32)
    b1 = b1_ref[...]
    fr = f_ref[...]
    theta = jnp.zeros_like(h)
    for l in range(L):
        s = seq_ref[l]
        g = jnp.dot(jnp.tanh(s), w1_ref[...], preferred_element_type=jnp.float32)
        xt = jax.nn.sigmoid(g + b1 + h)
        beta = jnp.sum(xt * fr, axis=1, keepdims=True)
        theta = theta + beta * s
    theta_ref[...] = theta


def _attention(seq_t, slen, w1p, b1p, w2p, fp):
    grid = (B // BB,)
    return pl.pallas_call(
        _attn_body,
        grid=grid,
        in_specs=[
            pl.BlockSpec((L, BB, DP), lambda i: (0, i, 0)),
            pl.BlockSpec((BB, 1), lambda i: (i, 0)),
            pl.BlockSpec((DP, DP), lambda i: (0, 0)),
            pl.BlockSpec((1, DP), lambda i: (0, 0)),
            pl.BlockSpec((DP, DP), lambda i: (0, 0)),
            pl.BlockSpec((1, DP), lambda i: (0, 0)),
        ],
        out_specs=pl.BlockSpec((BB, DP), lambda i: (i, 0)),
        out_shape=jax.ShapeDtypeStruct((B, DP), jnp.float32),
    )(seq_t, slen, w1p, b1p, w2p, fp)


def _scores_body(theta_ref, emb_ref, out_ref):
    out_ref[...] = jax.lax.dot_general(
        theta_ref[...], emb_ref[...],
        (((1,), (1,)), ((), ())),
        preferred_element_type=jnp.float32)


def _scores(theta, emb_m):
    grid = (NVP // VB,)
    return pl.pallas_call(
        _scores_body,
        grid=grid,
        in_specs=[
            pl.BlockSpec((B, DP), lambda j: (0, 0)),
            pl.BlockSpec((VB, DP), lambda j: (j, 0)),
        ],
        out_specs=pl.BlockSpec((B, VB), lambda j: (0, j)),
        out_shape=jax.ShapeDtypeStruct((B, NVP), jnp.float32),
    )(theta, emb_m)


def kernel(adj_rows, adj_cols, adj_vals, session_len, session_item,
           reversed_sess_item, mask, A, D, embedding, glu1_W, glu1_b,
           glu2_W, f):
    colslr = jnp.pad(
        jnp.bitwise_or(adj_cols, jnp.left_shift(adj_rows % R, 16)),
        (0, NNZP - NNZ))
    rowsp = jnp.pad(adj_rows, (0, NNZP - NNZ), constant_values=NB * R)
    valsp = jnp.pad(adj_vals, (0, NNZP - NNZ))
    bnd = jnp.searchsorted(rowsp, jnp.arange(NB + 1, dtype=_i32) * R)
    bidx = jnp.clip(jnp.arange(NW)[:, None] * (NB // NW)
                    + jnp.arange(16)[None, :], 0, NB)
    bounds = bnd.astype(_i32)[bidx]
    zeros = jnp.zeros((R, DP), _f32)

    t0 = jnp.zeros((TROWS, DP), _f32).at[:N_NODE, :EMB].set(embedding)

    t1 = _spmm(t0, colslr, valsp, bounds, zeros)
    t2 = _spmm(t1, colslr, valsp, bounds, zeros)
    t3 = _spmm(t2, colslr, valsp, bounds, zeros)

    rev_t = jnp.reshape(jnp.transpose(reversed_sess_item), (L * B,))
    idx_t = jnp.where(rev_t == 0, ZROW, rev_t - 1).astype(_i32)
    seq_flat = _gather4(t0, t1, t2, t3, idx_t)
    seq_t = jnp.reshape(seq_flat, (L, B, DP))

    w1p = jnp.pad(glu1_W, ((0, DP - EMB), (0, DP - EMB)))
    b1p = jnp.pad(glu1_b, (0, DP - EMB)).reshape(1, DP)
    w2p = jnp.pad(glu2_W, ((0, DP - EMB), (0, DP - EMB)))
    fp = jnp.pad(f[:, 0], (0, DP - EMB)).reshape(1, DP)

    theta = _attention(seq_t, session_len, w1p, b1p, w2p, fp)

    emb_m = jnp.pad(embedding, ((0, NVP - N_NODE), (0, DP - EMB)))
    scores = _scores(theta, emb_m)
    return scores[:, :N_NODE]


# speedup vs baseline: 2.3035x; 1.0390x over previous
import functools

import jax
import jax.numpy as jnp
from jax import lax
from jax.experimental import pallas as pl
from jax.experimental.pallas import tpu as pltpu
from jax.experimental.pallas import tpu_sc as plsc

N_NODE = 40727
EMB = 100
LAYERS = 3
B = 1024
L = 50
NNZ = 651632
DP = 128
VB = 512
NVP = 40960
BB = 256

NW = 32
NB = 128
R = 320
TROWS = NB * R
ZROW = N_NODE
KE = 128
NNZP = 651776
SK = 80

_f32 = jnp.float32
_i32 = jnp.int32


def _wid():
    return lax.axis_index("s") * 2 + lax.axis_index("c")


def _spmm_body(table, colslr, vals, bounds, zeros, out,
               acc, gath, ibuf, vbuf, gidx, bounds_v, sem_i, sem_g):
    w = _wid()
    pltpu.sync_copy(bounds, bounds_v)

    bw = bounds_v[w]

    def fire_idx(off, slot):
        pltpu.async_copy(colslr.at[pl.ds(off, KE)], ibuf.at[slot],
                         sem_i.at[slot])
        pltpu.async_copy(vals.at[pl.ds(off, KE)], vbuf.at[slot],
                         sem_i.at[slot])

    def wait_idx(off, slot):
        pltpu.make_async_copy(colslr.at[pl.ds(off, KE)], ibuf.at[slot],
                              sem_i.at[slot]).wait()
        pltpu.make_async_copy(vals.at[pl.ds(off, KE)], vbuf.at[slot],
                              sem_i.at[slot]).wait()

    def unpack_and_gather(slot, par):
        def up(g, _):
            sl = pl.ds(g * 16, 16)
            gidx[par, sl] = jnp.bitwise_and(ibuf[slot, sl], 0xFFFF)
            return 0
        lax.fori_loop(0, KE // 16, up, 0)
        pltpu.async_copy(table.at[gidx.at[par]], gath.at[par],
                         sem_g.at[par])

    def wait_gather(par):
        pltpu.make_async_copy(table.at[gidx.at[par]], gath.at[par],
                              sem_g.at[par]).wait()

    def do_block(k):
        b = w * (NB // NW) + k
        e0 = bw[k]
        e1 = bw[k + 1]
        ea = (e0 // 8) * 8
        nchunks = (e1 - ea + (KE - 1)) // KE
        pltpu.sync_copy(zeros, acc)

        for q in range(3):
            @pl.when(nchunks > q)
            def _():
                fire_idx(ea + q * KE, q)

        for q in range(2):
            @pl.when(nchunks > q)
            def _():
                wait_idx(ea + q * KE, q)
                unpack_and_gather(q, q)

        def chunk_body(j, _):
            off = ea + j * KE
            jp2 = j + 2
            jp3 = j + 3
            i0 = lax.rem(j, 4)
            i2 = lax.rem(jp2, 4)
            i3 = lax.rem(jp3, 4)
            p0 = lax.rem(j, 3)
            p2 = lax.rem(jp2, 3)

            @pl.when(jp3 < nchunks)
            def _():
                fire_idx(ea + jp3 * KE, i3)

            @pl.when(jp2 < nchunks)
            def _():
                wait_idx(ea + jp2 * KE, i2)
                unpack_and_gather(i2, p2)

            wait_gather(p0)

            i0full = jnp.full((16,), i0, _i32)

            def group_body(g, _):
                sl = pl.ds(g * 16, 16)
                e16 = lax.iota(_i32, 16) + (off + g * 16)
                ok16 = jnp.logical_and(e16 >= e0, e16 < e1)
                vbuf[i0, sl] = jnp.where(ok16, vbuf[i0, sl], 0.0)
                for i in range(16):
                    e = g * 16 + i
                    evec = jnp.full((16,), e, _i32)
                    v = plsc.load_gather(vbuf, [i0full, evec])
                    lrb = jnp.right_shift(
                        plsc.load_gather(ibuf, [i0full, evec]), 16)
                    prods = [gath[p0, e, pl.ds(d * 16, 16)] * v
                             for d in range(DP // 16)]
                    for d in range(DP // 16):
                        cidx = lax.iota(_i32, 16) + d * 16
                        plsc.addupdate_scatter(acc, [lrb, cidx], prods[d])
                return 0

            lax.fori_loop(0, KE // 16, group_body, 0)
            return 0

        lax.fori_loop(0, nchunks, chunk_body, 0)
        base = b * R
        pltpu.sync_copy(acc, out.at[pl.ds(base, R)])

    for k in range(NB // NW):
        do_block(k)


@functools.partial(
    pl.kernel,
    out_type=jax.ShapeDtypeStruct((TROWS, DP), _f32),
    mesh=plsc.VectorSubcoreMesh(core_axis_name="c", subcore_axis_name="s",
                                num_cores=2, num_subcores=16),
    compiler_params=pltpu.CompilerParams(needs_layout_passes=False),
    scratch_types=[
        pltpu.VMEM((R, DP), _f32),
        pltpu.VMEM((3, KE, DP), _f32),
        pltpu.VMEM((4, KE), _i32),
        pltpu.VMEM((4, KE), _f32),
        pltpu.VMEM((3, KE), _i32),
        pltpu.VMEM((NW, 16), _i32),
        pltpu.SemaphoreType.DMA((4,)),
        pltpu.SemaphoreType.DMA((3,)),
    ],
)
def _spmm(table, colslr, vals, bounds, zeros, out,
          acc, gath, ibuf, vbuf, gidx, bounds_v, sem_i, sem_g):
    _spmm_body(table, colslr, vals, bounds, zeros, out,
               acc, gath, ibuf, vbuf, gidx, bounds_v, sem_i, sem_g)


def _gather4_body(t0, t1, t2, t3, idx, out,
                  idx_v, g0, g1, g2, g3, ov, sem):
    w = _wid()
    per_w = (L * B) // NW
    nchunks = per_w // SK

    def chunk(j, _):
        off = w * per_w + j * SK
        pltpu.sync_copy(idx.at[pl.ds(off, SK)], idx_v)
        c0 = pltpu.async_copy(t0.at[idx_v], g0, sem)
        c1 = pltpu.async_copy(t1.at[idx_v], g1, sem)
        c2 = pltpu.async_copy(t2.at[idx_v], g2, sem)
        c3 = pltpu.async_copy(t3.at[idx_v], g3, sem)
        c0.wait(); c1.wait(); c2.wait(); c3.wait()

        def row(i, _):
            nd = DP // 16
            a = [g0[i, pl.ds(d * 16, 16)] + g1[i, pl.ds(d * 16, 16)]
                 for d in range(nd)]
            b = [g2[i, pl.ds(d * 16, 16)] + g3[i, pl.ds(d * 16, 16)]
                 for d in range(nd)]
            for d in range(nd):
                ov[i, pl.ds(d * 16, 16)] = (a[d] + b[d]) * 0.25
            return 0

        lax.fori_loop(0, SK, row, 0)
        pltpu.sync_copy(ov, out.at[pl.ds(off, SK)])
        return 0

    lax.fori_loop(0, nchunks, chunk, 0)


@functools.partial(
    pl.kernel,
    out_type=jax.ShapeDtypeStruct((L * B, DP), _f32),
    mesh=plsc.VectorSubcoreMesh(core_axis_name="c", subcore_axis_name="s",
                                num_cores=2, num_subcores=16),
    scratch_types=[
        pltpu.VMEM((SK,), _i32),
        pltpu.VMEM((SK, DP), _f32),
        pltpu.VMEM((SK, DP), _f32),
        pltpu.VMEM((SK, DP), _f32),
        pltpu.VMEM((SK, DP), _f32),
        pltpu.VMEM((SK, DP), _f32),
        pltpu.SemaphoreType.DMA,
    ],
)
def _gather4(t0, t1, t2, t3, idx, out, idx_v, g0, g1, g2, g3, ov, sem):
    _gather4_body(t0, t1, t2, t3, idx, out, idx_v, g0, g1, g2, g3, ov, sem)


def _attn_body(seq_ref, slen_ref, w1_ref, b1_ref, w2_ref, f_ref, theta_ref):
    z = seq_ref[0]
    for l in range(1, L):
        z = z + seq_ref[l]
    xs = z / slen_ref[...]
    h = jnp.dot(xs, w2_ref[...], preferred_element_type=jnp.float32)
    b1 = b1_ref[...]
    fr = f_ref[...]
    theta = jnp.zeros_like(h)
    for l in range(L):
        s = seq_ref[l]
        g = jnp.dot(jnp.tanh(s), w1_ref[...], preferred_element_type=jnp.float32)
        xt = jax.nn.sigmoid(g + b1 + h)
        beta = jnp.sum(xt * fr, axis=1, keepdims=True)
        theta = theta + beta * s
    theta_ref[...] = theta


def _attention(seq_t, slen, w1p, b1p, w2p, fp):
    grid = (B // BB,)
    return pl.pallas_call(
        _attn_body,
        grid=grid,
        in_specs=[
            pl.BlockSpec((L, BB, DP), lambda i: (0, i, 0)),
            pl.BlockSpec((BB, 1), lambda i: (i, 0)),
            pl.BlockSpec((DP, DP), lambda i: (0, 0)),
            pl.BlockSpec((1, DP), lambda i: (0, 0)),
            pl.BlockSpec((DP, DP), lambda i: (0, 0)),
            pl.BlockSpec((1, DP), lambda i: (0, 0)),
        ],
        out_specs=pl.BlockSpec((BB, DP), lambda i: (i, 0)),
        out_shape=jax.ShapeDtypeStruct((B, DP), jnp.float32),
    )(seq_t, slen, w1p, b1p, w2p, fp)


def _scores_body(theta_ref, emb_ref, out_ref):
    out_ref[...] = jax.lax.dot_general(
        theta_ref[...], emb_ref[...],
        (((1,), (1,)), ((), ())),
        preferred_element_type=jnp.float32)


def _scores(theta, emb_m):
    grid = (NVP // VB,)
    return pl.pallas_call(
        _scores_body,
        grid=grid,
        in_specs=[
            pl.BlockSpec((B, DP), lambda j: (0, 0)),
            pl.BlockSpec((VB, DP), lambda j: (j, 0)),
        ],
        out_specs=pl.BlockSpec((B, VB), lambda j: (0, j)),
        out_shape=jax.ShapeDtypeStruct((B, N_NODE), jnp.float32),
    )(theta, emb_m)


def _pad_body(emb_ref, out_ref):
    i = pl.program_id(0)
    rid = lax.broadcasted_iota(_i32, (VB, EMB), 0) + i * VB
    v = jnp.where(rid < N_NODE, emb_ref[...], 0.0)
    out_ref[...] = jnp.concatenate(
        [v, jnp.zeros((VB, DP - EMB), _f32)], axis=1)


def _pad_table(embedding):
    grid = (TROWS // VB,)
    return pl.pallas_call(
        _pad_body,
        grid=grid,
        in_specs=[pl.BlockSpec((VB, EMB), lambda i: (i, 0))],
        out_specs=pl.BlockSpec((VB, DP), lambda i: (i, 0)),
        out_shape=jax.ShapeDtypeStruct((TROWS, DP), jnp.float32),
    )(embedding)


def kernel(adj_rows, adj_cols, adj_vals, session_len, session_item,
           reversed_sess_item, mask, A, D, embedding, glu1_W, glu1_b,
           glu2_W, f):
    colslr = jnp.pad(
        jnp.bitwise_or(adj_cols, jnp.left_shift(adj_rows % R, 16)),
        (0, NNZP - NNZ))
    rowsp = jnp.pad(adj_rows, (0, NNZP - NNZ), constant_values=NB * R)
    valsp = jnp.pad(adj_vals, (0, NNZP - NNZ))
    bnd = jnp.searchsorted(rowsp, jnp.arange(NB + 1, dtype=_i32) * R)
    bidx = jnp.clip(jnp.arange(NW)[:, None] * (NB // NW)
                    + jnp.arange(16)[None, :], 0, NB)
    bounds = bnd.astype(_i32)[bidx]
    zeros = jnp.zeros((R, DP), _f32)

    t0 = _pad_table(embedding)

    t1 = _spmm(t0, colslr, valsp, bounds, zeros)
    t2 = _spmm(t1, colslr, valsp, bounds, zeros)
    t3 = _spmm(t2, colslr, valsp, bounds, zeros)

    rev_t = jnp.reshape(jnp.transpose(reversed_sess_item), (L * B,))
    idx_t = jnp.where(rev_t == 0, ZROW, rev_t - 1).astype(_i32)
    seq_flat = _gather4(t0, t1, t2, t3, idx_t)
    seq_t = jnp.reshape(seq_flat, (L, B, DP))

    w1p = jnp.pad(glu1_W, ((0, DP - EMB), (0, DP - EMB)))
    b1p = jnp.pad(glu1_b, (0, DP - EMB)).reshape(1, DP)
    w2p = jnp.pad(glu2_W, ((0, DP - EMB), (0, DP - EMB)))
    fp = jnp.pad(f[:, 0], (0, DP - EMB)).reshape(1, DP)

    theta = _attention(seq_t, session_len, w1p, b1p, w2p, fp)

    return _scores(theta, t0)


# speedup vs baseline: 2.3757x; 1.0313x over previous
import functools

import jax
import jax.numpy as jnp
from jax import lax
from jax.experimental import pallas as pl
from jax.experimental.pallas import tpu as pltpu
from jax.experimental.pallas import tpu_sc as plsc

N_NODE = 40727
EMB = 100
LAYERS = 3
B = 1024
L = 50
NNZ = 651632
DP = 128
VB = 512
NVP = 40960
BB = 256

NW = 32
NB = 128
R = 320
TROWS = NB * R
ZROW = N_NODE
KE = 128
NNZP = 651776
SK = 80

_f32 = jnp.float32
_i32 = jnp.int32


def _wid():
    return lax.axis_index("s") * 2 + lax.axis_index("c")


def _spmm_body(table, colslr, vals, bounds, zeros, out,
               acc, gath, ibuf, vbuf, gidx, bounds_v, sem_i, sem_g):
    w = _wid()
    pltpu.sync_copy(bounds, bounds_v)

    bw = bounds_v[w]

    def fire_idx(off, slot):
        pltpu.async_copy(colslr.at[pl.ds(off, KE)], ibuf.at[slot],
                         sem_i.at[slot])
        pltpu.async_copy(vals.at[pl.ds(off, KE)], vbuf.at[slot],
                         sem_i.at[slot])

    def wait_idx(off, slot):
        pltpu.make_async_copy(colslr.at[pl.ds(off, KE)], ibuf.at[slot],
                              sem_i.at[slot]).wait()
        pltpu.make_async_copy(vals.at[pl.ds(off, KE)], vbuf.at[slot],
                              sem_i.at[slot]).wait()

    def unpack_and_gather(slot, par):
        def up(g, _):
            sl = pl.ds(g * 16, 16)
            gidx[par, sl] = jnp.bitwise_and(ibuf[slot, sl], 0xFFFF)
            return 0
        lax.fori_loop(0, KE // 16, up, 0)
        pltpu.async_copy(table.at[gidx.at[par]], gath.at[par],
                         sem_g.at[par])

    def wait_gather(par):
        pltpu.make_async_copy(table.at[gidx.at[par]], gath.at[par],
                              sem_g.at[par]).wait()

    def do_block(k):
        b = w * (NB // NW) + k
        e0 = bw[k]
        e1 = bw[k + 1]
        ea = (e0 // 8) * 8
        nchunks = (e1 - ea + (KE - 1)) // KE
        pltpu.sync_copy(zeros, acc)

        for q in range(3):
            @pl.when(nchunks > q)
            def _():
                fire_idx(ea + q * KE, q)

        for q in range(2):
            @pl.when(nchunks > q)
            def _():
                wait_idx(ea + q * KE, q)
                unpack_and_gather(q, q)

        def chunk_body(j, _):
            off = ea + j * KE
            jp2 = j + 2
            jp3 = j + 3
            i0 = lax.rem(j, 4)
            i2 = lax.rem(jp2, 4)
            i3 = lax.rem(jp3, 4)
            p0 = lax.rem(j, 3)
            p2 = lax.rem(jp2, 3)

            @pl.when(jp3 < nchunks)
            def _():
                fire_idx(ea + jp3 * KE, i3)

            @pl.when(jp2 < nchunks)
            def _():
                wait_idx(ea + jp2 * KE, i2)
                unpack_and_gather(i2, p2)

            wait_gather(p0)

            i0full = jnp.full((16,), i0, _i32)

            def group_body(g, _):
                sl = pl.ds(g * 16, 16)
                e16 = lax.iota(_i32, 16) + (off + g * 16)
                ok16 = jnp.logical_and(e16 >= e0, e16 < e1)
                vbuf[i0, sl] = jnp.where(ok16, vbuf[i0, sl], 0.0)
                for i in range(16):
                    e = g * 16 + i
                    evec = jnp.full((16,), e, _i32)
                    v = plsc.load_gather(vbuf, [i0full, evec])
                    lrb = jnp.right_shift(
                        plsc.load_gather(ibuf, [i0full, evec]), 16)
                    prods = [gath[p0, e, pl.ds(d * 16, 16)] * v
                             for d in range(DP // 16)]
                    for d in range(DP // 16):
                        cidx = lax.iota(_i32, 16) + d * 16
                        plsc.addupdate_scatter(acc, [lrb, cidx], prods[d])
                return 0

            lax.fori_loop(0, KE // 16, group_body, 0)
            return 0

        lax.fori_loop(0, nchunks, chunk_body, 0)
        base = b * R
        pltpu.sync_copy(acc, out.at[pl.ds(base, R)])

    for k in range(NB // NW):
        do_block(k)


@functools.partial(
    pl.kernel,
    out_type=jax.ShapeDtypeStruct((TROWS, DP), _f32),
    mesh=plsc.VectorSubcoreMesh(core_axis_name="c", subcore_axis_name="s",
                                num_cores=2, num_subcores=16),
    compiler_params=pltpu.CompilerParams(needs_layout_passes=False),
    scratch_types=[
        pltpu.VMEM((R, DP), _f32),
        pltpu.VMEM((3, KE, DP), _f32),
        pltpu.VMEM((4, KE), _i32),
        pltpu.VMEM((4, KE), _f32),
        pltpu.VMEM((3, KE), _i32),
        pltpu.VMEM((NW, 16), _i32),
        pltpu.SemaphoreType.DMA((4,)),
        pltpu.SemaphoreType.DMA((3,)),
    ],
)
def _spmm(table, colslr, vals, bounds, zeros, out,
          acc, gath, ibuf, vbuf, gidx, bounds_v, sem_i, sem_g):
    _spmm_body(table, colslr, vals, bounds, zeros, out,
               acc, gath, ibuf, vbuf, gidx, bounds_v, sem_i, sem_g)


def _gather4_body(t0, t1, t2, t3, idx, out,
                  idx_v, g0, g1, g2, g3, ov, sem_i, sem_g, sem_o):
    w = _wid()
    per_w = (L * B) // NW
    nch = per_w // SK
    tabs = (t0, t1, t2, t3)
    gbufs = (g0, g1, g2, g3)

    def off_of(j):
        return w * per_w + j * SK

    def fire_idx(j, slot):
        pltpu.async_copy(idx.at[pl.ds(off_of(j), SK)], idx_v.at[slot],
                         sem_i.at[slot])

    def wait_idx(j, slot):
        pltpu.make_async_copy(idx.at[pl.ds(off_of(j), SK)], idx_v.at[slot],
                              sem_i.at[slot]).wait()

    def fire_gathers(slot, par):
        for t, gb in zip(tabs, gbufs):
            pltpu.async_copy(t.at[idx_v.at[slot]], gb.at[par],
                             sem_g.at[par])

    def wait_gathers(slot, par):
        for t, gb in zip(tabs, gbufs):
            pltpu.make_async_copy(t.at[idx_v.at[slot]], gb.at[par],
                                  sem_g.at[par]).wait()

    def fire_out(j, par):
        pltpu.async_copy(ov.at[par], out.at[pl.ds(off_of(j), SK)],
                         sem_o.at[par])

    def wait_out(j, par):
        pltpu.make_async_copy(ov.at[par], out.at[pl.ds(off_of(j), SK)],
                              sem_o.at[par]).wait()

    fire_idx(0, 0)
    fire_idx(1, 1)
    wait_idx(0, 0)
    fire_gathers(0, 0)

    def chunk(j, _):
        jp1 = j + 1
        jp2 = j + 2
        s2 = lax.rem(jp2, 3)
        s1 = lax.rem(jp1, 3)
        s0 = lax.rem(j, 3)
        p1 = lax.rem(jp1, 2)
        p0 = lax.rem(j, 2)

        @pl.when(jp2 < nch)
        def _():
            fire_idx(jp2, s2)

        @pl.when(jp1 < nch)
        def _():
            wait_idx(jp1, s1)
            fire_gathers(s1, p1)

        wait_gathers(s0, p0)

        @pl.when(j >= 2)
        def _():
            wait_out(j - 2, p0)

        def row(i, _):
            nd = DP // 16
            a = [g0[p0, i, pl.ds(d * 16, 16)] + g1[p0, i, pl.ds(d * 16, 16)]
                 for d in range(nd)]
            b = [g2[p0, i, pl.ds(d * 16, 16)] + g3[p0, i, pl.ds(d * 16, 16)]
                 for d in range(nd)]
            for d in range(nd):
                ov[p0, i, pl.ds(d * 16, 16)] = (a[d] + b[d]) * 0.25
            return 0

        lax.fori_loop(0, SK, row, 0)
        fire_out(j, p0)
        return 0

    lax.fori_loop(0, nch, chunk, 0)
    wait_out(nch - 2, 0)
    wait_out(nch - 1, 1)


@functools.partial(
    pl.kernel,
    out_type=jax.ShapeDtypeStruct((L * B, DP), _f32),
    mesh=plsc.VectorSubcoreMesh(core_axis_name="c", subcore_axis_name="s",
                                num_cores=2, num_subcores=16),
    compiler_params=pltpu.CompilerParams(needs_layout_passes=False),
    scratch_types=[
        pltpu.VMEM((3, SK), _i32),
        pltpu.VMEM((2, SK, DP), _f32),
        pltpu.VMEM((2, SK, DP), _f32),
        pltpu.VMEM((2, SK, DP), _f32),
        pltpu.VMEM((2, SK, DP), _f32),
        pltpu.VMEM((2, SK, DP), _f32),
        pltpu.SemaphoreType.DMA((3,)),
        pltpu.SemaphoreType.DMA((2,)),
        pltpu.SemaphoreType.DMA((2,)),
    ],
)
def _gather4(t0, t1, t2, t3, idx, out, idx_v, g0, g1, g2, g3, ov,
             sem_i, sem_g, sem_o):
    _gather4_body(t0, t1, t2, t3, idx, out, idx_v, g0, g1, g2, g3, ov,
                  sem_i, sem_g, sem_o)


def _attn_body(seq_ref, slen_ref, w1_ref, b1_ref, w2_ref, f_ref, theta_ref):
    z = seq_ref[0]
    for l in range(1, L):
        z = z + seq_ref[l]
    xs = z / slen_ref[...]
    h = jnp.dot(xs, w2_ref[...], preferred_element_type=jnp.float32)
    b1 = b1_ref[...]
    fr = f_ref[...]
    theta = jnp.zeros_like(h)
    for l in range(L):
        s = seq_ref[l]
        g = jnp.dot(jnp.tanh(s), w1_ref[...], preferred_element_type=jnp.float32)
        xt = jax.nn.sigmoid(g + b1 + h)
        beta = jnp.sum(xt * fr, axis=1, keepdims=True)
        theta = theta + beta * s
    theta_ref[...] = theta


def _attention(seq_t, slen, w1p, b1p, w2p, fp):
    grid = (B // BB,)
    return pl.pallas_call(
        _attn_body,
        grid=grid,
        in_specs=[
            pl.BlockSpec((L, BB, DP), lambda i: (0, i, 0)),
            pl.BlockSpec((BB, 1), lambda i: (i, 0)),
            pl.BlockSpec((DP, DP), lambda i: (0, 0)),
            pl.BlockSpec((1, DP), lambda i: (0, 0)),
            pl.BlockSpec((DP, DP), lambda i: (0, 0)),
            pl.BlockSpec((1, DP), lambda i: (0, 0)),
        ],
        out_specs=pl.BlockSpec((BB, DP), lambda i: (i, 0)),
        out_shape=jax.ShapeDtypeStruct((B, DP), jnp.float32),
    )(seq_t, slen, w1p, b1p, w2p, fp)


def _scores_body(theta_ref, emb_ref, out_ref):
    out_ref[...] = jax.lax.dot_general(
        theta_ref[...], emb_ref[...],
        (((1,), (1,)), ((), ())),
        preferred_element_type=jnp.float32)


def _scores(theta, emb_m):
    grid = (NVP // VB,)
    return pl.pallas_call(
        _scores_body,
        grid=grid,
        in_specs=[
            pl.BlockSpec((B, DP), lambda j: (0, 0)),
            pl.BlockSpec((VB, DP), lambda j: (j, 0)),
        ],
        out_specs=pl.BlockSpec((B, VB), lambda j: (0, j)),
        out_shape=jax.ShapeDtypeStruct((B, N_NODE), jnp.float32),
    )(theta, emb_m)


def _pad_body(emb_ref, out_ref):
    i = pl.program_id(0)
    rid = lax.broadcasted_iota(_i32, (VB, EMB), 0) + i * VB
    v = jnp.where(rid < N_NODE, emb_ref[...], 0.0)
    out_ref[...] = jnp.concatenate(
        [v, jnp.zeros((VB, DP - EMB), _f32)], axis=1)


def _pad_table(embedding):
    grid = (TROWS // VB,)
    return pl.pallas_call(
        _pad_body,
        grid=grid,
        in_specs=[pl.BlockSpec((VB, EMB), lambda i: (i, 0))],
        out_specs=pl.BlockSpec((VB, DP), lambda i: (i, 0)),
        out_shape=jax.ShapeDtypeStruct((TROWS, DP), jnp.float32),
    )(embedding)


def kernel(adj_rows, adj_cols, adj_vals, session_len, session_item,
           reversed_sess_item, mask, A, D, embedding, glu1_W, glu1_b,
           glu2_W, f):
    colslr = jnp.pad(
        jnp.bitwise_or(adj_cols, jnp.left_shift(adj_rows % R, 16)),
        (0, NNZP - NNZ))
    rowsp = jnp.pad(adj_rows, (0, NNZP - NNZ), constant_values=NB * R)
    valsp = jnp.pad(adj_vals, (0, NNZP - NNZ))
    bnd = jnp.searchsorted(rowsp, jnp.arange(NB + 1, dtype=_i32) * R)
    bidx = jnp.clip(jnp.arange(NW)[:, None] * (NB // NW)
                    + jnp.arange(16)[None, :], 0, NB)
    bounds = bnd.astype(_i32)[bidx]
    zeros = jnp.zeros((R, DP), _f32)

    t0 = _pad_table(embedding)

    t1 = _spmm(t0, colslr, valsp, bounds, zeros)
    t2 = _spmm(t1, colslr, valsp, bounds, zeros)
    t3 = _spmm(t2, colslr, valsp, bounds, zeros)

    rev_t = jnp.reshape(jnp.transpose(reversed_sess_item), (L * B,))
    idx_t = jnp.where(rev_t == 0, ZROW, rev_t - 1).astype(_i32)
    seq_flat = _gather4(t0, t1, t2, t3, idx_t)
    seq_t = jnp.reshape(seq_flat, (L, B, DP))

    w1p = jnp.pad(glu1_W, ((0, DP - EMB), (0, DP - EMB)))
    b1p = jnp.pad(glu1_b, (0, DP - EMB)).reshape(1, DP)
    w2p = jnp.pad(glu2_W, ((0, DP - EMB), (0, DP - EMB)))
    fp = jnp.pad(f[:, 0], (0, DP - EMB)).reshape(1, DP)

    theta = _attention(seq_t, session_len, w1p, b1p, w2p, fp)

    return _scores(theta, t0)
